# Initial kernel scaffold; baseline (speedup 1.0000x reference)
import jax
import jax.numpy as jnp
from jax.experimental import pallas as pl


def kernel(x, W1, g1, b1, W2, g2, b2, W3, g3, b3, W4, g4, b4, W5, g5, b5):
    raise NotImplementedError("write your pallas kernel here")


# speedup vs baseline: 11.2414x; 11.2414x over previous
import functools

import jax
import jax.numpy as jnp
from jax import lax
from jax.experimental import pallas as pl
from jax.experimental.pallas import tpu as pltpu
from jax.experimental.pallas import tpu_sc as plsc

EPSV = 1e-5
KNB = 20
_NC, _NS = 2, 16
_NW = _NC * _NS


def _knn_kernel(xr_ref, xa_ref, sqr_ref, sqa_ref, idx_ref, *, k, n):
    xr = xr_ref[0]
    xa = xa_ref[0]
    R = xr.shape[0]
    base = pl.program_id(0) * n
    G = jax.lax.dot_general(xr, xa, (((1,), (1,)), ((), ())),
                            preferred_element_type=jnp.float32)
    d = (2.0 * G - sqr_ref[0]) - sqa_ref[0]
    iota = jax.lax.broadcasted_iota(jnp.int32, (R, n), 1)
    kiota = jax.lax.broadcasted_iota(jnp.int32, (R, k), 1)
    idx = jnp.zeros((R, k), jnp.int32)
    for r in range(k):
        rowmax = jnp.max(d, axis=1, keepdims=True)
        cand = jnp.where(d == rowmax, iota, n)
        jmin = jnp.min(cand, axis=1, keepdims=True)
        idx = jnp.where(kiota == r, jmin, idx)
        d = jnp.where(cand == jmin, -jnp.inf, d)
    idx_ref[0] = idx + base


def _knn_idx(xt, xx, block_rows=256):
    B, N, C = xt.shape
    sqr = xx[:, :, None]
    sqa = xx[:, None, :]
    return pl.pallas_call(
        functools.partial(_knn_kernel, k=KNB, n=N),
        grid=(B, N // block_rows),
        in_specs=[
            pl.BlockSpec((1, block_rows, C), lambda b, i: (b, i, 0)),
            pl.BlockSpec((1, N, C), lambda b, i: (b, 0, 0)),
            pl.BlockSpec((1, block_rows, 1), lambda b, i: (b, i, 0)),
            pl.BlockSpec((1, 1, N), lambda b, i: (b, 0, 0)),
        ],
        out_specs=pl.BlockSpec((1, block_rows, KNB), lambda b, i: (b, i, 0)),
        out_shape=jax.ShapeDtypeStruct((B, N, KNB), jnp.int32),
    )(xt, xt, sqr, sqa)


def _sc_gather_body(tab_hbm, idx_hbm, out_hbm, idx_v, rows_v, sem,
                    *, tot_idx, nsub):
    wid = lax.axis_index("s") * _NC + lax.axis_index("c")
    span = tot_idx // _NW
    rows_per_it = nsub * 128
    base_row = wid * (span // 128)

    def step(t, carry):
        r0 = base_row + t * nsub
        pltpu.sync_copy(idx_hbm.at[pl.ds(r0, nsub)], idx_v)
        cps = [
            pltpu.async_copy(tab_hbm.at[idx_v.at[j]],
                             rows_v.at[pl.ds(j * 128, 128)], sem)
            for j in range(nsub)
        ]
        for cp in cps:
            cp.wait()
        pltpu.sync_copy(rows_v, out_hbm.at[pl.ds(r0 * 128, rows_per_it)])
        return carry

    lax.fori_loop(0, span // rows_per_it, step, 0, unroll=False)


def _sc_gather(tab, idx2d, nsub=8):
    tot, C = tab.shape
    tot_idx = idx2d.shape[0] * 128
    mesh = plsc.VectorSubcoreMesh(core_axis_name="c", subcore_axis_name="s")
    fn = pl.kernel(
        functools.partial(_sc_gather_body, tot_idx=tot_idx, nsub=nsub),
        out_type=jax.ShapeDtypeStruct((tot_idx, C), jnp.float32),
        mesh=mesh,
        scratch_types=[
            pltpu.VMEM((nsub, 128), jnp.int32),
            pltpu.VMEM((nsub * 128, C), jnp.float32),
            pltpu.SemaphoreType.DMA,
        ],
        compiler_params=pltpu.CompilerParams(use_tc_tiling_on_sc=False),
    )
    return fn(tab, idx2d)


def _conv_kernel(xg_ref, xi_ref, w_ref, s_ref, b_ref, out_ref, *, k):
    xi = xi_ref[0]
    C = xi.shape[1]
    xg = xg_ref[0][:, :C]
    RK = xg.shape[0]
    R = RK // k
    o = w_ref.shape[1]
    xie = jnp.broadcast_to(xi[:, None, :], (R, k, C)).reshape(RK, C)
    f = jnp.concatenate([xg - xie, xie], axis=1)
    h = jax.lax.dot_general(f, w_ref[...], (((1,), (0,)), ((), ())),
                            preferred_element_type=jnp.float32)
    y = h * s_ref[...] + b_ref[...]
    y = jnp.where(y > 0, y, 0.2 * y)
    out_ref[0] = jnp.max(y.reshape(R, k, o), axis=1)


def _conv_max(xg, xt, W, g, b, block_rows=256):
    B, N, C = xt.shape
    Cg = xg.shape[2]
    o = W.shape[0]
    w = jnp.transpose(W)
    scale = (g / jnp.sqrt(1.0 + EPSV))[None, :]
    bias = b[None, :]
    return pl.pallas_call(
        functools.partial(_conv_kernel, k=KNB),
        grid=(B, N // block_rows),
        in_specs=[
            pl.BlockSpec((1, block_rows * KNB, Cg), lambda bb, i: (bb, i, 0)),
            pl.BlockSpec((1, block_rows, C), lambda bb, i: (bb, i, 0)),
            pl.BlockSpec((2 * C, o), lambda bb, i: (0, 0)),
            pl.BlockSpec((1, o), lambda bb, i: (0, 0)),
            pl.BlockSpec((1, o), lambda bb, i: (0, 0)),
        ],
        out_specs=pl.BlockSpec((1, block_rows, o), lambda bb, i: (bb, i, 0)),
        out_shape=jax.ShapeDtypeStruct((B, N, o), jnp.float32),
    )(xg, xt, w, scale, bias)


def _edge_layer(xc, W, g, b):
    B, C, N = xc.shape
    xt = jnp.transpose(xc, (0, 2, 1))
    xx = jnp.sum(xc ** 2, axis=1)
    idx = _knn_idx(xt, xx)
    tab = xt.reshape(B * N, C)
    Cp = max(C, 16)
    if Cp != C:
        tab = jnp.concatenate(
            [tab, jnp.zeros((B * N, Cp - C), tab.dtype)], axis=1)
    idx2d = idx.reshape(B * N * KNB // 128, 128)
    xg = _sc_gather(tab, idx2d)
    xg = xg.reshape(B, N * KNB, Cp)
    out = _conv_max(xg, xt, W, g, b)
    return jnp.transpose(out, (0, 2, 1))


def _final_kernel(x1_ref, x2_ref, x3_ref, x4_ref, w_ref, s_ref, b_ref, out_ref):
    xc = jnp.concatenate(
        [x1_ref[0], x2_ref[0], x3_ref[0], x4_ref[0]], axis=1)
    h = jax.lax.dot_general(xc, w_ref[...], (((1,), (0,)), ((), ())),
                            preferred_element_type=jnp.float32)
    y = h * s_ref[...] + b_ref[...]
    y = jnp.where(y > 0, y, 0.2 * y)
    pmax = jnp.max(y, axis=0, keepdims=True)
    pavg = jnp.sum(y, axis=0, keepdims=True) * (1.0 / y.shape[0])
    out_ref[0] = jnp.concatenate([pmax, pavg], axis=1)


def _final_layer(x1, x2, x3, x4, W5, g5, b5):
    B, _, N = x1.shape
    xs = [jnp.transpose(v, (0, 2, 1)) for v in (x1, x2, x3, x4)]
    w = jnp.transpose(W5)
    scale = (g5 / jnp.sqrt(1.0 + EPSV))[None, :]
    bias = b5[None, :]
    specs = [pl.BlockSpec((1, N, v.shape[2]), lambda bb: (bb, 0, 0))
             for v in xs]
    out = pl.pallas_call(
        _final_kernel,
        grid=(B,),
        in_specs=specs + [
            pl.BlockSpec((128, 64), lambda bb: (0, 0)),
            pl.BlockSpec((1, 64), lambda bb: (0, 0)),
            pl.BlockSpec((1, 64), lambda bb: (0, 0)),
        ],
        out_specs=pl.BlockSpec((1, 1, 128), lambda bb: (bb, 0, 0)),
        out_shape=jax.ShapeDtypeStruct((B, 1, 128), jnp.float32),
    )(*xs, w, scale, bias)
    return out[:, 0, :]


def kernel(x, W1, g1, b1, W2, g2, b2, W3, g3, b3, W4, g4, b4, W5, g5, b5):
    x1 = _edge_layer(x, W1, g1, b1)
    x2 = _edge_layer(x1, W2, g2, b2)
    x3 = _edge_layer(x2, W3, g3, b3)
    x4 = _edge_layer(x3, W4, g4, b4)
    return _final_layer(x1, x2, x3, x4, W5, g5, b5)


# speedup vs baseline: 14.7345x; 1.3107x over previous
import functools

import jax
import jax.numpy as jnp
from jax import lax
from jax.experimental import pallas as pl
from jax.experimental.pallas import tpu as pltpu
from jax.experimental.pallas import tpu_sc as plsc

EPSV = 1e-5
KNB = 20
_NC, _NS = 2, 16
_NW = _NC * _NS


def _knn_kernel(xr_ref, xa_ref, sqr_ref, sqa_ref, idx_ref, *, k, n):
    xr = xr_ref[0]
    xa = xa_ref[0]
    R = xr.shape[0]
    base = pl.program_id(0) * n
    G = jax.lax.dot_general(xr, xa, (((1,), (1,)), ((), ())),
                            preferred_element_type=jnp.float32)
    d = (2.0 * G - sqr_ref[0]) - sqa_ref[0]
    m = jax.lax.bitcast_convert_type(d, jnp.int32)
    mono = jnp.where(m < 0, m ^ jnp.int32(0x7FFFFFFF), m)
    key = (mono & jnp.int32(~(n - 1))) | (
        (n - 1) - jax.lax.broadcasted_iota(jnp.int32, (R, n), 1))
    kiota = jax.lax.broadcasted_iota(jnp.int32, (R, k), 1)
    idx = jnp.zeros((R, k), jnp.int32)
    for r in range(k):
        rowmax = jnp.max(key, axis=1, keepdims=True)
        col = (n - 1) - (rowmax & (n - 1))
        idx = jnp.where(kiota == r, col, idx)
        key = jnp.where(key == rowmax, jnp.int32(-0x80000000), key)
    idx_ref[0] = idx + base


def _knn_idx(xt, xx, block_rows=256):
    B, N, C = xt.shape
    sqr = xx[:, :, None]
    sqa = xx[:, None, :]
    return pl.pallas_call(
        functools.partial(_knn_kernel, k=KNB, n=N),
        grid=(B, N // block_rows),
        in_specs=[
            pl.BlockSpec((1, block_rows, C), lambda b, i: (b, i, 0)),
            pl.BlockSpec((1, N, C), lambda b, i: (b, 0, 0)),
            pl.BlockSpec((1, block_rows, 1), lambda b, i: (b, i, 0)),
            pl.BlockSpec((1, 1, N), lambda b, i: (b, 0, 0)),
        ],
        out_specs=pl.BlockSpec((1, block_rows, KNB), lambda b, i: (b, i, 0)),
        out_shape=jax.ShapeDtypeStruct((B, N, KNB), jnp.int32),
    )(xt, xt, sqr, sqa)


def _sc_gather_body(tab_hbm, idx_hbm, out_hbm, idx_v, rows_v, sem,
                    *, tot_idx, nsub):
    wid = lax.axis_index("s") * _NC + lax.axis_index("c")
    span = tot_idx // _NW
    rows_per_it = nsub * 128
    base_row = wid * (span // 128)

    def step(t, carry):
        r0 = base_row + t * nsub
        pltpu.sync_copy(idx_hbm.at[pl.ds(r0, nsub)], idx_v)
        cps = [
            pltpu.async_copy(tab_hbm.at[idx_v.at[j]],
                             rows_v.at[pl.ds(j * 128, 128)], sem)
            for j in range(nsub)
        ]
        for cp in cps:
            cp.wait()
        pltpu.sync_copy(rows_v, out_hbm.at[pl.ds(r0 * 128, rows_per_it)])
        return carry

    lax.fori_loop(0, span // rows_per_it, step, 0, unroll=False)


def _sc_gather(tab, idx2d, nsub=8):
    tot, C = tab.shape
    tot_idx = idx2d.shape[0] * 128
    mesh = plsc.VectorSubcoreMesh(core_axis_name="c", subcore_axis_name="s")
    fn = pl.kernel(
        functools.partial(_sc_gather_body, tot_idx=tot_idx, nsub=nsub),
        out_type=jax.ShapeDtypeStruct((tot_idx, C), jnp.float32),
        mesh=mesh,
        scratch_types=[
            pltpu.VMEM((nsub, 128), jnp.int32),
            pltpu.VMEM((nsub * 128, C), jnp.float32),
            pltpu.SemaphoreType.DMA,
        ],
        compiler_params=pltpu.CompilerParams(use_tc_tiling_on_sc=False),
    )
    return fn(tab, idx2d)


def _conv_kernel(xg_ref, xi_ref, w_ref, s_ref, b_ref, out_ref, *, k):
    xi = xi_ref[0]
    C = xi.shape[1]
    xg = xg_ref[0][:, :C]
    RK = xg.shape[0]
    R = RK // k
    o = w_ref.shape[1]
    xie = jnp.broadcast_to(xi[:, None, :], (R, k, C)).reshape(RK, C)
    f = jnp.concatenate([xg - xie, xie], axis=1)
    h = jax.lax.dot_general(f, w_ref[...], (((1,), (0,)), ((), ())),
                            preferred_element_type=jnp.float32)
    y = h * s_ref[...] + b_ref[...]
    y = jnp.where(y > 0, y, 0.2 * y)
    out_ref[0] = jnp.max(y.reshape(R, k, o), axis=1)


def _conv_max(xg, xt, W, g, b, block_rows=256):
    B, N, C = xt.shape
    Cg = xg.shape[2]
    o = W.shape[0]
    w = jnp.transpose(W)
    scale = (g / jnp.sqrt(1.0 + EPSV))[None, :]
    bias = b[None, :]
    return pl.pallas_call(
        functools.partial(_conv_kernel, k=KNB),
        grid=(B, N // block_rows),
        in_specs=[
            pl.BlockSpec((1, block_rows * KNB, Cg), lambda bb, i: (bb, i, 0)),
            pl.BlockSpec((1, block_rows, C), lambda bb, i: (bb, i, 0)),
            pl.BlockSpec((2 * C, o), lambda bb, i: (0, 0)),
            pl.BlockSpec((1, o), lambda bb, i: (0, 0)),
            pl.BlockSpec((1, o), lambda bb, i: (0, 0)),
        ],
        out_specs=pl.BlockSpec((1, block_rows, o), lambda bb, i: (bb, i, 0)),
        out_shape=jax.ShapeDtypeStruct((B, N, o), jnp.float32),
    )(xg, xt, w, scale, bias)


def _edge_layer(xc, W, g, b):
    B, C, N = xc.shape
    xt = jnp.transpose(xc, (0, 2, 1))
    xx = jnp.sum(xc ** 2, axis=1)
    idx = _knn_idx(xt, xx)
    tab = xt.reshape(B * N, C)
    Cp = max(C, 16)
    if Cp != C:
        tab = jnp.concatenate(
            [tab, jnp.zeros((B * N, Cp - C), tab.dtype)], axis=1)
    idx2d = idx.reshape(B * N * KNB // 128, 128)
    xg = _sc_gather(tab, idx2d)
    xg = xg.reshape(B, N * KNB, Cp)
    out = _conv_max(xg, xt, W, g, b)
    return jnp.transpose(out, (0, 2, 1))


def _final_kernel(x1_ref, x2_ref, x3_ref, x4_ref, w_ref, s_ref, b_ref, out_ref):
    xc = jnp.concatenate(
        [x1_ref[0], x2_ref[0], x3_ref[0], x4_ref[0]], axis=1)
    h = jax.lax.dot_general(xc, w_ref[...], (((1,), (0,)), ((), ())),
                            preferred_element_type=jnp.float32)
    y = h * s_ref[...] + b_ref[...]
    y = jnp.where(y > 0, y, 0.2 * y)
    pmax = jnp.max(y, axis=0, keepdims=True)
    pavg = jnp.sum(y, axis=0, keepdims=True) * (1.0 / y.shape[0])
    out_ref[0] = jnp.concatenate([pmax, pavg], axis=1)


def _final_layer(x1, x2, x3, x4, W5, g5, b5):
    B, _, N = x1.shape
    xs = [jnp.transpose(v, (0, 2, 1)) for v in (x1, x2, x3, x4)]
    w = jnp.transpose(W5)
    scale = (g5 / jnp.sqrt(1.0 + EPSV))[None, :]
    bias = b5[None, :]
    specs = [pl.BlockSpec((1, N, v.shape[2]), lambda bb: (bb, 0, 0))
             for v in xs]
    out = pl.pallas_call(
        _final_kernel,
        grid=(B,),
        in_specs=specs + [
            pl.BlockSpec((128, 64), lambda bb: (0, 0)),
            pl.BlockSpec((1, 64), lambda bb: (0, 0)),
            pl.BlockSpec((1, 64), lambda bb: (0, 0)),
        ],
        out_specs=pl.BlockSpec((1, 1, 128), lambda bb: (bb, 0, 0)),
        out_shape=jax.ShapeDtypeStruct((B, 1, 128), jnp.float32),
    )(*xs, w, scale, bias)
    return out[:, 0, :]


def kernel(x, W1, g1, b1, W2, g2, b2, W3, g3, b3, W4, g4, b4, W5, g5, b5):
    x1 = _edge_layer(x, W1, g1, b1)
    x2 = _edge_layer(x1, W2, g2, b2)
    x3 = _edge_layer(x2, W3, g3, b3)
    x4 = _edge_layer(x3, W4, g4, b4)
    return _final_layer(x1, x2, x3, x4, W5, g5, b5)
